# baseline (speedup 1.0000x reference)
import functools
import jax
import jax.numpy as jnp
from jax import lax
from jax.experimental import pallas as pl
from jax.experimental.pallas import tpu as pltpu
from jax.experimental.pallas import tpu_sc as plsc

NC = 2
NS = 16
NW = NC * NS
EB = 128
F = 128

TC_BLK = 512
N_PAD = 10240
RPS = N_PAD // NS
BLK_A = 56
BLK_B = 104


def _spmm_body(u_hbm, src_hbm, dst_hbm, zeros_hbm, out_hbm,
               idx_s, idx_d, rows, sem, isem, acc, *, blk_a, blk_b):
    c = lax.axis_index("c")
    s = lax.axis_index("s")
    nb = jnp.where(c == 0, blk_a, blk_b)
    start = jnp.where(c == 0, s * blk_a, NS * blk_a + s * blk_b)

    pltpu.sync_copy(zeros_hbm.at[pl.ds(s * RPS, RPS)],
                    acc.at[pl.ds(s * RPS, RPS)])
    @pl.when(c == 0)
    def _():
        pltpu.sync_copy(src_hbm.at[pl.ds(s * blk_a, blk_a)],
                        idx_s.at[pl.ds(0, blk_a)])

    @pl.when(c == 1)
    def _():
        pltpu.sync_copy(src_hbm.at[pl.ds(NS * blk_a + s * blk_b, blk_b)],
                        idx_s.at[pl.ds(0, blk_b)])

    pltpu.sync_copy(dst_hbm.at[start], idx_d.at[0])
    plsc.subcore_barrier()

    pltpu.async_copy(u_hbm.at[idx_s.at[0]], rows.at[0], sem)

    def body(j, carry):
        p = jnp.bitwise_and(j, 1)
        pltpu.make_async_copy(u_hbm.at[idx_s.at[j]], rows.at[p], sem).wait()

        @pl.when(j + 1 < nb)
        def _():
            pltpu.async_copy(u_hbm.at[idx_s.at[j + 1]], rows.at[1 - p], sem)

        @pl.when(j > 0)
        def _():
            pltpu.make_async_copy(dst_hbm.at[start + j], idx_d.at[p],
                                  isem).wait()

        @pl.when(j + 1 < nb)
        def _():
            pltpu.async_copy(dst_hbm.at[start + j + 1], idx_d.at[1 - p], isem)

        pltpu.sync_copy(rows.at[p], acc.at[idx_d.at[p]], add=True)
        return carry

    lax.fori_loop(0, nb, body, 0)
    plsc.subcore_barrier()
    pltpu.sync_copy(acc.at[pl.ds(s * RPS, RPS)],
                    out_hbm.at[c, pl.ds(s * RPS, RPS)])


def _make_spmm(blk_a, blk_b):
    return pl.kernel(
        functools.partial(_spmm_body, blk_a=blk_a, blk_b=blk_b),
        out_type=jax.ShapeDtypeStruct((NC, N_PAD, F), jnp.float32),
        mesh=plsc.VectorSubcoreMesh(core_axis_name="c", subcore_axis_name="s",
                                 num_cores=NC, num_subcores=NS),
        scratch_types=[
            pltpu.VMEM((max(blk_a, blk_b), EB), jnp.int32),
            pltpu.VMEM((2, EB), jnp.int32),
            pltpu.VMEM((2, EB, F), jnp.float32),
            pltpu.SemaphoreType.DMA,
            pltpu.SemaphoreType.DMA,
            pltpu.VMEM_SHARED((N_PAD, F), jnp.float32),
        ],
    )


def _deg_body(src_hbm, dst_hbm, zeros_hbm, ones_hbm, out_hbm,
              idx, ones_v, acc):
    c = lax.axis_index("c")
    s = lax.axis_index("s")
    nb = src_hbm.shape[0] // NS

    pltpu.sync_copy(zeros_hbm.at[pl.ds(s * RPS, RPS)],
                    acc.at[pl.ds(s * RPS, RPS)])
    pltpu.sync_copy(ones_hbm, ones_v)

    @pl.when(c == 0)
    def _():
        pltpu.sync_copy(src_hbm.at[pl.ds(s * nb, nb)], idx)

    @pl.when(c == 1)
    def _():
        pltpu.sync_copy(dst_hbm.at[pl.ds(s * nb, nb)], idx)

    plsc.subcore_barrier()

    def body(j, carry):
        pltpu.sync_copy(ones_v, acc.at[idx.at[j]], add=True)
        return carry

    lax.fori_loop(0, nb, body, 0)
    plsc.subcore_barrier()
    pltpu.sync_copy(acc.at[pl.ds(s * RPS, RPS)],
                    out_hbm.at[c, pl.ds(s * RPS, RPS)])


def _make_deg(tot_blk):
    return pl.kernel(
        _deg_body,
        out_type=jax.ShapeDtypeStruct((NC, N_PAD, F), jnp.float32),
        mesh=plsc.VectorSubcoreMesh(core_axis_name="c", subcore_axis_name="s",
                                 num_cores=NC, num_subcores=NS),
        scratch_types=[
            pltpu.VMEM((tot_blk // NS, EB), jnp.int32),
            pltpu.VMEM((EB, F), jnp.float32),
            pltpu.VMEM_SHARED((N_PAD, F), jnp.float32),
        ],
    )


def _dinv(deg_blk, kind):
    d = deg_blk[kind, :, 0]
    return (1.0 / jnp.sqrt(jnp.maximum(d, 1.0)))[:, None]


def _prep_body(x_ref, w_ref, deg_ref, s_out, u1_out):
    sval = lax.dot_general(x_ref[...], w_ref[...], (((1,), (0,)), ((), ())),
                           precision=lax.Precision.HIGHEST,
                           preferred_element_type=jnp.float32)
    s_out[...] = sval
    u1_out[...] = (4.0 * sval[:, 2 * F:3 * F]) * _dinv(deg_ref[...], 0)


def _combine_body(s_ref, p_ref, deg_ref, u_out, *, a, ca, b, cb):
    deg = deg_ref[...]
    w = -_dinv(deg, 1) * (p_ref[0] + p_ref[1])
    t = a * s_ref[:, ca:ca + F] + w
    if b != 0.0:
        t = t + b * s_ref[:, cb:cb + F]
    u_out[...] = _dinv(deg, 0) * t


def _final_body(s_ref, p_ref, deg_ref, bias_ref, o_ref):
    w = -_dinv(deg_ref[...], 1) * (p_ref[0] + p_ref[1])
    o_ref[...] = w - s_ref[:, F:2 * F] + bias_ref[...]


_GRID = N_PAD // TC_BLK

_deg_spec = pl.BlockSpec((2, TC_BLK, F), lambda i: (0, i, 0))
_s_spec = pl.BlockSpec((TC_BLK, 3 * F), lambda i: (i, 0))
_p_spec = pl.BlockSpec((NC, TC_BLK, F), lambda i: (0, i, 0))
_u_spec = pl.BlockSpec((TC_BLK, F), lambda i: (i, 0))

_prep = pl.pallas_call(
    _prep_body,
    grid=(_GRID,),
    in_specs=[_u_spec,
              pl.BlockSpec((F, 3 * F), lambda i: (0, 0)),
              _deg_spec],
    out_specs=[_s_spec, _u_spec],
    out_shape=[jax.ShapeDtypeStruct((N_PAD, 3 * F), jnp.float32),
               jax.ShapeDtypeStruct((N_PAD, F), jnp.float32)],
)


def _make_combine(a, ca, b, cb):
    return pl.pallas_call(
        functools.partial(_combine_body, a=a, ca=ca, b=b, cb=cb),
        grid=(_GRID,),
        in_specs=[_s_spec, _p_spec, _deg_spec],
        out_specs=_u_spec,
        out_shape=jax.ShapeDtypeStruct((N_PAD, F), jnp.float32),
    )


_combine2 = _make_combine(2.0, F, 0.0, 0)
_combine3 = _make_combine(1.0, 0, -3.0, 2 * F)

_final = pl.pallas_call(
    _final_body,
    grid=(_GRID,),
    in_specs=[_s_spec, _p_spec, _deg_spec,
              pl.BlockSpec((1, F), lambda i: (0, 0))],
    out_specs=_u_spec,
    out_shape=jax.ShapeDtypeStruct((N_PAD, F), jnp.float32),
)


def kernel(x, edge_index, weight, bias):
    n, f_in = x.shape
    e = edge_index.shape[1]
    tot_blk = NS * (BLK_A + BLK_B)
    assert tot_blk * EB >= e
    e_pad = tot_blk * EB

    pad = jnp.full((e_pad - e,), n, dtype=jnp.int32)
    src3 = jnp.concatenate([edge_index[0], pad]).reshape(tot_blk, EB)
    dst3 = jnp.concatenate([edge_index[1], pad]).reshape(tot_blk, EB)

    x_pad = jnp.concatenate(
        [x, jnp.zeros((N_PAD - n, f_in), jnp.float32)], axis=0)
    wcat = jnp.concatenate([weight[0], weight[1], weight[2]], axis=1)

    z128 = jnp.zeros((N_PAD, F), jnp.float32)
    o128 = jnp.ones((EB, F), jnp.float32)

    deg = _make_deg(tot_blk)(src3, dst3, z128, o128)
    s_mat, u1 = _prep(x_pad, wcat, deg)

    spmm = _make_spmm(BLK_A, BLK_B)
    p1 = spmm(u1, src3, dst3, z128)
    u2 = _combine2(s_mat, p1, deg)
    p2 = spmm(u2, src3, dst3, z128)
    u3 = _combine3(s_mat, p2, deg)
    p3 = spmm(u3, src3, dst3, z128)

    out = _final(s_mat, p3, deg, bias.reshape(1, F))
    return out[:n]


# speedup vs baseline: 1.6032x; 1.6032x over previous
import functools
import jax
import jax.numpy as jnp
from jax import lax
from jax.experimental import pallas as pl
from jax.experimental.pallas import tpu as pltpu
from jax.experimental.pallas import tpu_sc as plsc

NC = 2
NS = 16
NW = NC * NS
EB = 128
F = 128

TC_BLK = 512
N_PAD = 10240
RPS = N_PAD // NS


def _spmm_body(u_hbm, src_hbm, dst_hbm, zeros_hbm, out_hbm,
               idx_s, idx_d, rows, sem, isem, acc):
    c = lax.axis_index("c")
    s = lax.axis_index("s")
    wid = s * NC + c
    nblk = src_hbm.shape[1]

    pltpu.sync_copy(zeros_hbm.at[pl.ds(s * RPS, RPS)],
                    acc.at[pl.ds(s * RPS, RPS)])
    pltpu.sync_copy(src_hbm.at[wid], idx_s)
    pltpu.sync_copy(dst_hbm.at[wid, 0], idx_d.at[0])
    plsc.subcore_barrier()

    pltpu.async_copy(u_hbm.at[idx_s.at[0]], rows.at[0], sem)

    def body(j, carry):
        p = jnp.bitwise_and(j, 1)
        pltpu.make_async_copy(u_hbm.at[idx_s.at[j]], rows.at[p], sem).wait()

        @pl.when(j + 1 < nblk)
        def _():
            pltpu.async_copy(u_hbm.at[idx_s.at[j + 1]], rows.at[1 - p], sem)

        @pl.when(j > 0)
        def _():
            pltpu.make_async_copy(dst_hbm.at[wid, j], idx_d.at[p], isem).wait()

        @pl.when(j + 1 < nblk)
        def _():
            pltpu.async_copy(dst_hbm.at[wid, j + 1], idx_d.at[1 - p], isem)

        pltpu.sync_copy(rows.at[p], acc.at[idx_d.at[p]], add=True)
        return carry

    lax.fori_loop(0, nblk, body, 0)
    plsc.subcore_barrier()
    pltpu.sync_copy(acc.at[pl.ds(s * RPS, RPS)],
                    out_hbm.at[c, pl.ds(s * RPS, RPS)])


def _make_spmm(nblk):
    return pl.kernel(
        _spmm_body,
        out_type=jax.ShapeDtypeStruct((NC, N_PAD, F), jnp.float32),
        mesh=plsc.VectorSubcoreMesh(core_axis_name="c", subcore_axis_name="s",
                                 num_cores=NC, num_subcores=NS),
        scratch_types=[
            pltpu.VMEM((nblk, EB), jnp.int32),
            pltpu.VMEM((2, EB), jnp.int32),
            pltpu.VMEM((2, EB, F), jnp.float32),
            pltpu.SemaphoreType.DMA,
            pltpu.SemaphoreType.DMA,
            pltpu.VMEM_SHARED((N_PAD, F), jnp.float32),
        ],
    )


def _deg_body(src_hbm, dst_hbm, zeros_hbm, ones_hbm, out_hbm,
              idx, ones_v, acc):
    c = lax.axis_index("c")
    s = lax.axis_index("s")
    nblk = src_hbm.shape[1]

    pltpu.sync_copy(zeros_hbm.at[pl.ds(s * RPS, RPS)],
                    acc.at[pl.ds(s * RPS, RPS)])
    pltpu.sync_copy(ones_hbm, ones_v)

    @pl.when(c == 0)
    def _():
        pltpu.sync_copy(src_hbm.at[2 * s], idx.at[pl.ds(0, nblk)])
        pltpu.sync_copy(src_hbm.at[2 * s + 1], idx.at[pl.ds(nblk, nblk)])

    @pl.when(c == 1)
    def _():
        pltpu.sync_copy(dst_hbm.at[2 * s], idx.at[pl.ds(0, nblk)])
        pltpu.sync_copy(dst_hbm.at[2 * s + 1], idx.at[pl.ds(nblk, nblk)])

    plsc.subcore_barrier()

    def body(j, carry):
        pltpu.sync_copy(ones_v, acc.at[idx.at[j]], add=True)
        return carry

    lax.fori_loop(0, 2 * nblk, body, 0)
    plsc.subcore_barrier()
    pltpu.sync_copy(acc.at[pl.ds(s * RPS, RPS)],
                    out_hbm.at[c, pl.ds(s * RPS, RPS)])


def _make_deg(nblk):
    return pl.kernel(
        _deg_body,
        out_type=jax.ShapeDtypeStruct((NC, N_PAD, F), jnp.float32),
        mesh=plsc.VectorSubcoreMesh(core_axis_name="c", subcore_axis_name="s",
                                 num_cores=NC, num_subcores=NS),
        scratch_types=[
            pltpu.VMEM((2 * nblk, EB), jnp.int32),
            pltpu.VMEM((EB, F), jnp.float32),
            pltpu.VMEM_SHARED((N_PAD, F), jnp.float32),
        ],
    )


def _dinv(deg_blk, kind):
    d = deg_blk[kind, :, 0]
    return (1.0 / jnp.sqrt(jnp.maximum(d, 1.0)))[:, None]


def _matmul_body(x_ref, w_ref, s_out):
    s_out[...] = lax.dot_general(x_ref[...], w_ref[...],
                                 (((1,), (0,)), ((), ())),
                                 precision=lax.Precision.HIGHEST,
                                 preferred_element_type=jnp.float32)


def _scale1_body(s_ref, deg_ref, u1_out):
    u1_out[...] = (4.0 * s_ref[:, 2 * F:3 * F]) * _dinv(deg_ref[...], 0)


def _combine_body(s_ref, p_ref, deg_ref, u_out, *, a, ca, b, cb):
    deg = deg_ref[...]
    w = -_dinv(deg, 1) * (p_ref[0] + p_ref[1])
    t = a * s_ref[:, ca:ca + F] + w
    if b != 0.0:
        t = t + b * s_ref[:, cb:cb + F]
    u_out[...] = _dinv(deg, 0) * t


def _final_body(s_ref, p_ref, deg_ref, bias_ref, o_ref):
    w = -_dinv(deg_ref[...], 1) * (p_ref[0] + p_ref[1])
    o_ref[...] = w - s_ref[:, F:2 * F] + bias_ref[...]


_GRID = N_PAD // TC_BLK

_deg_spec = pl.BlockSpec((2, TC_BLK, F), lambda i: (0, i, 0))
_s_spec = pl.BlockSpec((TC_BLK, 3 * F), lambda i: (i, 0))
_p_spec = pl.BlockSpec((NC, TC_BLK, F), lambda i: (0, i, 0))
_u_spec = pl.BlockSpec((TC_BLK, F), lambda i: (i, 0))

_matmul = pl.pallas_call(
    _matmul_body,
    grid=(_GRID,),
    in_specs=[_u_spec, pl.BlockSpec((F, 3 * F), lambda i: (0, 0))],
    out_specs=_s_spec,
    out_shape=jax.ShapeDtypeStruct((N_PAD, 3 * F), jnp.float32),
)

_scale1 = pl.pallas_call(
    _scale1_body,
    grid=(_GRID,),
    in_specs=[_s_spec, _deg_spec],
    out_specs=_u_spec,
    out_shape=jax.ShapeDtypeStruct((N_PAD, F), jnp.float32),
)


def _make_combine(a, ca, b, cb):
    return pl.pallas_call(
        functools.partial(_combine_body, a=a, ca=ca, b=b, cb=cb),
        grid=(_GRID,),
        in_specs=[_s_spec, _p_spec, _deg_spec],
        out_specs=_u_spec,
        out_shape=jax.ShapeDtypeStruct((N_PAD, F), jnp.float32),
    )


_combine2 = _make_combine(2.0, F, 0.0, 0)
_combine3 = _make_combine(1.0, 0, -3.0, 2 * F)

_final = pl.pallas_call(
    _final_body,
    grid=(_GRID,),
    in_specs=[_s_spec, _p_spec, _deg_spec,
              pl.BlockSpec((1, F), lambda i: (0, 0))],
    out_specs=_u_spec,
    out_shape=jax.ShapeDtypeStruct((N_PAD, F), jnp.float32),
)


def kernel(x, edge_index, weight, bias):
    n, f_in = x.shape
    e = edge_index.shape[1]
    nblk = -(-e // (NW * EB))
    e_pad = NW * nblk * EB

    pad = jnp.full((e_pad - e,), n, dtype=jnp.int32)
    src3 = jnp.concatenate([edge_index[0], pad]).reshape(NW, nblk, EB)
    dst3 = jnp.concatenate([edge_index[1], pad]).reshape(NW, nblk, EB)

    x_pad = jnp.concatenate(
        [x, jnp.zeros((N_PAD - n, f_in), jnp.float32)], axis=0)
    wcat = jnp.concatenate([weight[0], weight[1], weight[2]], axis=1)

    z128 = jnp.zeros((N_PAD, F), jnp.float32)
    o128 = jnp.ones((EB, F), jnp.float32)

    deg = _make_deg(nblk)(src3, dst3, z128, o128)
    s_mat = _matmul(x_pad, wcat)
    u1 = _scale1(s_mat, deg)

    spmm = _make_spmm(nblk)
    p1 = spmm(u1, src3, dst3, z128)
    u2 = _combine2(s_mat, p1, deg)
    p2 = spmm(u2, src3, dst3, z128)
    u3 = _combine3(s_mat, p2, deg)
    p3 = spmm(u3, src3, dst3, z128)

    out = _final(s_mat, p3, deg, bias.reshape(1, F))
    return out[:n]


# speedup vs baseline: 1.8958x; 1.1825x over previous
import functools
import jax
import jax.numpy as jnp
from jax import lax
from jax.experimental import pallas as pl
from jax.experimental.pallas import tpu as pltpu
from jax.experimental.pallas import tpu_sc as plsc

NC = 2
NS = 16
NW = NC * NS
EB = 128
F = 128

TC_BLK = 512
N_PAD = 10240
RPS = N_PAD // NS


def _spmm_body(u_hbm, src_hbm, dst_hbm, zeros_hbm, out_hbm,
               idx_s, idx_d, rows, sem, isem, acc):
    c = lax.axis_index("c")
    s = lax.axis_index("s")
    wid = s * NC + c
    nblk = src_hbm.shape[1]

    pltpu.sync_copy(zeros_hbm.at[pl.ds(s * RPS, RPS)],
                    acc.at[pl.ds(s * RPS, RPS)])
    pltpu.sync_copy(src_hbm.at[wid], idx_s)
    pltpu.sync_copy(dst_hbm.at[wid, 0], idx_d.at[0])
    plsc.subcore_barrier()

    pltpu.async_copy(u_hbm.at[idx_s.at[0]], rows.at[0], sem)

    def body(j, carry):
        p = jnp.bitwise_and(j, 1)
        pltpu.make_async_copy(u_hbm.at[idx_s.at[j]], rows.at[p], sem).wait()

        @pl.when(j + 1 < nblk)
        def _():
            pltpu.async_copy(u_hbm.at[idx_s.at[j + 1]], rows.at[1 - p], sem)

        @pl.when(j > 0)
        def _():
            pltpu.make_async_copy(dst_hbm.at[wid, j], idx_d.at[p], isem).wait()

        @pl.when(j + 1 < nblk)
        def _():
            pltpu.async_copy(dst_hbm.at[wid, j + 1], idx_d.at[1 - p], isem)

        pltpu.sync_copy(rows.at[p], acc.at[idx_d.at[p]], add=True)
        return carry

    lax.fori_loop(0, nblk, body, 0)
    plsc.subcore_barrier()
    pltpu.sync_copy(acc.at[pl.ds(s * RPS, RPS)],
                    out_hbm.at[c, pl.ds(s * RPS, RPS)])


def _make_spmm(nblk):
    return pl.kernel(
        _spmm_body,
        out_type=jax.ShapeDtypeStruct((NC, N_PAD, F), jnp.float32),
        mesh=plsc.VectorSubcoreMesh(core_axis_name="c", subcore_axis_name="s",
                                 num_cores=NC, num_subcores=NS),
        scratch_types=[
            pltpu.VMEM((nblk, EB), jnp.int32),
            pltpu.VMEM((2, EB), jnp.int32),
            pltpu.VMEM((2, EB, F), jnp.float32),
            pltpu.SemaphoreType.DMA,
            pltpu.SemaphoreType.DMA,
            pltpu.VMEM_SHARED((N_PAD, F), jnp.float32),
        ],
    )


def _deg_body(src_hbm, dst_hbm, zeros_hbm, ones_hbm, out_hbm,
              idx, ones_v, acc):
    c = lax.axis_index("c")
    s = lax.axis_index("s")
    nblk = src_hbm.shape[1]

    pltpu.sync_copy(zeros_hbm.at[pl.ds(s * RPS, RPS)],
                    acc.at[pl.ds(s * RPS, RPS)])
    pltpu.sync_copy(ones_hbm, ones_v)

    @pl.when(c == 0)
    def _():
        pltpu.sync_copy(src_hbm.at[2 * s], idx.at[pl.ds(0, nblk)])
        pltpu.sync_copy(src_hbm.at[2 * s + 1], idx.at[pl.ds(nblk, nblk)])

    @pl.when(c == 1)
    def _():
        pltpu.sync_copy(dst_hbm.at[2 * s], idx.at[pl.ds(0, nblk)])
        pltpu.sync_copy(dst_hbm.at[2 * s + 1], idx.at[pl.ds(nblk, nblk)])

    plsc.subcore_barrier()

    def body(j, carry):
        pltpu.sync_copy(ones_v, acc.at[idx.at[j]], add=True)
        return carry

    lax.fori_loop(0, 2 * nblk, body, 0)
    plsc.subcore_barrier()
    pltpu.sync_copy(acc.at[pl.ds(s * RPS, RPS)],
                    out_hbm.at[c, pl.ds(s * RPS, RPS)])


def _make_deg(nblk):
    return pl.kernel(
        _deg_body,
        out_type=jax.ShapeDtypeStruct((NC, N_PAD, F), jnp.float32),
        mesh=plsc.VectorSubcoreMesh(core_axis_name="c", subcore_axis_name="s",
                                 num_cores=NC, num_subcores=NS),
        scratch_types=[
            pltpu.VMEM((2 * nblk, EB), jnp.int32),
            pltpu.VMEM((EB, F), jnp.float32),
            pltpu.VMEM_SHARED((N_PAD, F), jnp.float32),
        ],
    )


def _dinv(deg_blk, kind):
    d = deg_blk[kind, :, 0]
    return (1.0 / jnp.sqrt(jnp.maximum(d, 1.0)))[:, None]


def _prep_body(x_ref, w_ref, deg_ref, s_out, u1_out):
    sval = lax.dot_general(x_ref[...], w_ref[...], (((1,), (0,)), ((), ())),
                           precision=lax.Precision.HIGHEST,
                           preferred_element_type=jnp.float32)
    s_out[...] = sval
    u1_out[...] = (4.0 * sval[:, 2 * F:3 * F]) * _dinv(deg_ref[...], 0)


def _combine_body(s_ref, p_ref, deg_ref, u_out, *, a, ca, b, cb):
    deg = deg_ref[...]
    w = -_dinv(deg, 1) * (p_ref[0] + p_ref[1])
    t = a * s_ref[:, ca:ca + F] + w
    if b != 0.0:
        t = t + b * s_ref[:, cb:cb + F]
    u_out[...] = _dinv(deg, 0) * t


def _final_body(s_ref, p_ref, deg_ref, bias_ref, o_ref):
    w = -_dinv(deg_ref[...], 1) * (p_ref[0] + p_ref[1])
    o_ref[...] = w - s_ref[:, F:2 * F] + bias_ref[...]


_GRID = N_PAD // TC_BLK

_deg_spec = pl.BlockSpec((2, TC_BLK, F), lambda i: (0, i, 0))
_s_spec = pl.BlockSpec((TC_BLK, 3 * F), lambda i: (i, 0))
_p_spec = pl.BlockSpec((NC, TC_BLK, F), lambda i: (0, i, 0))
_u_spec = pl.BlockSpec((TC_BLK, F), lambda i: (i, 0))

_prep = pl.pallas_call(
    _prep_body,
    grid=(_GRID,),
    in_specs=[_u_spec,
              pl.BlockSpec((F, 3 * F), lambda i: (0, 0)),
              _deg_spec],
    out_specs=[_s_spec, _u_spec],
    out_shape=[jax.ShapeDtypeStruct((N_PAD, 3 * F), jnp.float32),
               jax.ShapeDtypeStruct((N_PAD, F), jnp.float32)],
)


def _make_combine(a, ca, b, cb):
    return pl.pallas_call(
        functools.partial(_combine_body, a=a, ca=ca, b=b, cb=cb),
        grid=(_GRID,),
        in_specs=[_s_spec, _p_spec, _deg_spec],
        out_specs=_u_spec,
        out_shape=jax.ShapeDtypeStruct((N_PAD, F), jnp.float32),
    )


_combine2 = _make_combine(2.0, F, 0.0, 0)
_combine3 = _make_combine(1.0, 0, -3.0, 2 * F)

_final = pl.pallas_call(
    _final_body,
    grid=(_GRID,),
    in_specs=[_s_spec, _p_spec, _deg_spec,
              pl.BlockSpec((1, F), lambda i: (0, 0))],
    out_specs=_u_spec,
    out_shape=jax.ShapeDtypeStruct((N_PAD, F), jnp.float32),
)


def kernel(x, edge_index, weight, bias):
    n, f_in = x.shape
    e = edge_index.shape[1]
    nblk = -(-e // (NW * EB))
    e_pad = NW * nblk * EB

    pad = jnp.full((e_pad - e,), n, dtype=jnp.int32)
    src3 = jnp.concatenate([edge_index[0], pad]).reshape(NW, nblk, EB)
    dst3 = jnp.concatenate([edge_index[1], pad]).reshape(NW, nblk, EB)

    x_pad = jnp.concatenate(
        [x, jnp.zeros((N_PAD - n, f_in), jnp.float32)], axis=0)
    wcat = jnp.concatenate([weight[0], weight[1], weight[2]], axis=1)

    z128 = jnp.zeros((N_PAD, F), jnp.float32)
    o128 = jnp.ones((EB, F), jnp.float32)

    deg = _make_deg(nblk)(src3, dst3, z128, o128)
    s_mat, u1 = _prep(x_pad, wcat, deg)

    spmm = _make_spmm(nblk)
    p1 = spmm(u1, src3, dst3, z128)
    u2 = _combine2(s_mat, p1, deg)
    p2 = spmm(u2, src3, dst3, z128)
    u3 = _combine3(s_mat, p2, deg)
    p3 = spmm(u3, src3, dst3, z128)

    out = _final(s_mat, p3, deg, bias.reshape(1, F))
    return out[:n]
